# baseline (speedup 1.0000x reference)
import functools

import jax
import jax.numpy as jnp
from jax import lax
from jax.experimental import pallas as pl
from jax.experimental.pallas import tpu as pltpu
from jax.experimental.pallas import tpu_sc as plsc

_BATCH = 4096
_PIXELS = 12288


def _sc_gather(idx, a_flat, b_flat):
    info = plsc.get_sparse_core_info()
    nw = info.num_cores * info.num_subcores
    b_per_w = _BATCH // nw

    mesh = plsc.VectorSubcoreMesh(core_axis_name="c", subcore_axis_name="s")

    @functools.partial(
        pl.kernel,
        out_type=(
            jax.ShapeDtypeStruct((_BATCH,), jnp.float32),
            jax.ShapeDtypeStruct((_BATCH,), jnp.float32),
        ),
        mesh=mesh,
        scratch_types=[
            pltpu.VMEM((b_per_w,), jnp.int32),
            pltpu.VMEM((b_per_w,), jnp.float32),
            pltpu.VMEM((b_per_w,), jnp.float32),
            pltpu.SemaphoreType.DMA,
            pltpu.SemaphoreType.DMA,
        ],
        compiler_params=pltpu.CompilerParams(disable_bounds_checks=True),
    )
    def sc_kernel(idx_hbm, a_hbm, b_hbm, scale_hbm, off_hbm,
                  idx_v, a_v, b_v, sem_a, sem_b):
        wid = lax.axis_index("s") * info.num_cores + lax.axis_index("c")
        base = wid * b_per_w
        pltpu.sync_copy(idx_hbm.at[pl.ds(base, b_per_w)], idx_v)
        cp_a = pltpu.async_copy(a_hbm.at[idx_v], a_v, sem_a)
        cp_b = pltpu.async_copy(b_hbm.at[idx_v], b_v, sem_b)
        cp_a.wait()
        cp_b.wait()
        for i in range(b_per_w // 16):
            sl = pl.ds(i * 16, 16)
            a_v[sl] = jnp.exp(a_v[sl])
        pltpu.sync_copy(a_v, scale_hbm.at[pl.ds(base, b_per_w)])
        pltpu.sync_copy(b_v, off_hbm.at[pl.ds(base, b_per_w)])

    return sc_kernel(idx, a_flat, b_flat)


def _tc_apply(scale, offset, image):
    rows = 256
    cols = _PIXELS // 2
    grid = (_BATCH // rows, 2)

    def body(s_ref, o_ref, img_ref, out_ref):
        s = s_ref[...].reshape(rows, 1)
        o = o_ref[...].reshape(rows, 1)
        out_ref[...] = jnp.clip(s * img_ref[...] + o, 0.0, 1.0)

    return pl.pallas_call(
        body,
        grid=grid,
        in_specs=[
            pl.BlockSpec((rows,), lambda i, j: (i,)),
            pl.BlockSpec((rows,), lambda i, j: (i,)),
            pl.BlockSpec((rows, cols), lambda i, j: (i, j)),
        ],
        out_specs=pl.BlockSpec((rows, cols), lambda i, j: (i, j)),
        out_shape=jax.ShapeDtypeStruct((_BATCH, _PIXELS), jnp.float32),
        compiler_params=pltpu.CompilerParams(
            vmem_limit_bytes=62 * 1024 * 1024),
    )(scale, offset, image)


def kernel(idx, image, exposure_a, exposure_b):
    scale, offset = _sc_gather(idx, exposure_a.reshape(-1),
                               exposure_b.reshape(-1))
    return _tc_apply(scale, offset, image)


# speedup vs baseline: 1.0082x; 1.0082x over previous
import functools

import jax
import jax.numpy as jnp
from jax import lax
from jax.experimental import pallas as pl
from jax.experimental.pallas import tpu as pltpu
from jax.experimental.pallas import tpu_sc as plsc

_BATCH = 4096
_PIXELS = 12288


def _sc_gather(idx, a_flat, b_flat):
    info = plsc.get_sparse_core_info()
    nw = info.num_cores * info.num_subcores
    b_per_w = _BATCH // nw

    mesh = plsc.VectorSubcoreMesh(core_axis_name="c", subcore_axis_name="s")

    @functools.partial(
        pl.kernel,
        out_type=(
            jax.ShapeDtypeStruct((_BATCH,), jnp.float32),
            jax.ShapeDtypeStruct((_BATCH,), jnp.float32),
        ),
        mesh=mesh,
        scratch_types=[
            pltpu.VMEM((b_per_w,), jnp.int32),
            pltpu.VMEM((b_per_w,), jnp.float32),
            pltpu.VMEM((b_per_w,), jnp.float32),
            pltpu.SemaphoreType.DMA,
            pltpu.SemaphoreType.DMA,
        ],
        compiler_params=pltpu.CompilerParams(disable_bounds_checks=True),
    )
    def sc_kernel(idx_hbm, a_hbm, b_hbm, scale_hbm, off_hbm,
                  idx_v, a_v, b_v, sem_a, sem_b):
        wid = lax.axis_index("s") * info.num_cores + lax.axis_index("c")
        base = wid * b_per_w
        pltpu.sync_copy(idx_hbm.at[pl.ds(base, b_per_w)], idx_v)
        cp_a = pltpu.async_copy(a_hbm.at[idx_v], a_v, sem_a)
        cp_b = pltpu.async_copy(b_hbm.at[idx_v], b_v, sem_b)
        cp_a.wait()
        cp_b.wait()
        for i in range(b_per_w // 16):
            sl = pl.ds(i * 16, 16)
            a_v[sl] = jnp.exp(a_v[sl])
        pltpu.sync_copy(a_v, scale_hbm.at[pl.ds(base, b_per_w)])
        pltpu.sync_copy(b_v, off_hbm.at[pl.ds(base, b_per_w)])

    return sc_kernel(idx, a_flat, b_flat)


_CH = 128
_NB = 4
_NCH = _BATCH // _CH


def _tc_apply(scale, offset, image):
    def body(s_ref, o_ref, img_hbm, out_hbm, in_buf, out_buf, in_sem, out_sem):
        def in_copy(i, slot):
            return pltpu.make_async_copy(
                img_hbm.at[pl.ds(i * _CH, _CH)], in_buf.at[slot],
                in_sem.at[slot])

        def out_copy(i, slot):
            return pltpu.make_async_copy(
                out_buf.at[slot], out_hbm.at[pl.ds(i * _CH, _CH)],
                out_sem.at[slot])

        for k in range(_NB - 1):
            in_copy(k, k).start()

        def step(i, _):
            slot = lax.rem(i, _NB)
            nxt = i + _NB - 1

            @pl.when(nxt < _NCH)
            def _():
                in_copy(nxt, lax.rem(nxt, _NB)).start()

            in_copy(i, slot).wait()

            @pl.when(i >= _NB)
            def _():
                out_copy(i - _NB, slot).wait()

            s = s_ref[pl.ds(pl.multiple_of(i * _CH, _CH), _CH)].reshape(_CH, 1)
            o = o_ref[pl.ds(pl.multiple_of(i * _CH, _CH), _CH)].reshape(_CH, 1)
            out_buf[slot] = jnp.clip(s * in_buf[slot] + o, 0.0, 1.0)
            out_copy(i, slot).start()
            return 0

        lax.fori_loop(0, _NCH, step, 0)
        for k in range(_NB):
            i = _NCH - _NB + k
            out_copy(i, i % _NB).wait()

    return pl.pallas_call(
        body,
        in_specs=[
            pl.BlockSpec(memory_space=pltpu.VMEM),
            pl.BlockSpec(memory_space=pltpu.VMEM),
            pl.BlockSpec(memory_space=pl.ANY),
        ],
        out_specs=pl.BlockSpec(memory_space=pl.ANY),
        out_shape=jax.ShapeDtypeStruct((_BATCH, _PIXELS), jnp.float32),
        scratch_shapes=[
            pltpu.VMEM((_NB, _CH, _PIXELS), jnp.float32),
            pltpu.VMEM((_NB, _CH, _PIXELS), jnp.float32),
            pltpu.SemaphoreType.DMA((_NB,)),
            pltpu.SemaphoreType.DMA((_NB,)),
        ],
        compiler_params=pltpu.CompilerParams(
            vmem_limit_bytes=62 * 1024 * 1024),
    )(scale, offset, image)


def kernel(idx, image, exposure_a, exposure_b):
    scale, offset = _sc_gather(idx, exposure_a.reshape(-1),
                               exposure_b.reshape(-1))
    return _tc_apply(scale, offset, image)


# speedup vs baseline: 1.0092x; 1.0010x over previous
import functools

import jax
import jax.numpy as jnp
from jax import lax
from jax.experimental import pallas as pl
from jax.experimental.pallas import tpu as pltpu
from jax.experimental.pallas import tpu_sc as plsc

_BATCH = 4096
_PIXELS = 12288


def _sc_gather(idx, a_flat, b_flat):
    info = plsc.get_sparse_core_info()
    nw = info.num_cores * info.num_subcores
    b_per_w = _BATCH // nw

    mesh = plsc.VectorSubcoreMesh(core_axis_name="c", subcore_axis_name="s")

    @functools.partial(
        pl.kernel,
        out_type=(
            jax.ShapeDtypeStruct((_BATCH,), jnp.float32),
            jax.ShapeDtypeStruct((_BATCH,), jnp.float32),
        ),
        mesh=mesh,
        scratch_types=[
            pltpu.VMEM((b_per_w,), jnp.int32),
            pltpu.VMEM((b_per_w,), jnp.float32),
            pltpu.VMEM((b_per_w,), jnp.float32),
            pltpu.SemaphoreType.DMA,
            pltpu.SemaphoreType.DMA,
        ],
        compiler_params=pltpu.CompilerParams(disable_bounds_checks=True),
    )
    def sc_kernel(idx_hbm, a_hbm, b_hbm, scale_hbm, off_hbm,
                  idx_v, a_v, b_v, sem_a, sem_b):
        wid = lax.axis_index("s") * info.num_cores + lax.axis_index("c")
        base = wid * b_per_w
        pltpu.sync_copy(idx_hbm.at[pl.ds(base, b_per_w)], idx_v)
        cp_a = pltpu.async_copy(a_hbm.at[idx_v], a_v, sem_a)
        cp_b = pltpu.async_copy(b_hbm.at[idx_v], b_v, sem_b)
        cp_a.wait()
        cp_b.wait()
        for i in range(b_per_w // 16):
            sl = pl.ds(i * 16, 16)
            a_v[sl] = jnp.exp(a_v[sl])
        pltpu.sync_copy(a_v, scale_hbm.at[pl.ds(base, b_per_w)])
        pltpu.sync_copy(b_v, off_hbm.at[pl.ds(base, b_per_w)])

    return sc_kernel(idx, a_flat, b_flat)


_CH = 128
_NB = 4
_NCH = _BATCH // _CH


def _tc_apply(scale, offset, image):
    def body(s_ref, o_ref, img_hbm, out_hbm, in_buf, out_buf, in_sem, out_sem):
        _H = _CH // 2

        def _in_half(i, slot, h):
            return pltpu.make_async_copy(
                img_hbm.at[pl.ds(i * _CH + h * _H, _H)],
                in_buf.at[slot, pl.ds(h * _H, _H)], in_sem.at[slot])

        def _out_half(i, slot, h):
            return pltpu.make_async_copy(
                out_buf.at[slot, pl.ds(h * _H, _H)],
                out_hbm.at[pl.ds(i * _CH + h * _H, _H)], out_sem.at[slot])

        class _Pair:
            def __init__(self, mk, i, slot):
                self.c = [mk(i, slot, 0), mk(i, slot, 1)]

            def start(self):
                self.c[0].start()
                self.c[1].start()

            def wait(self):
                self.c[0].wait()
                self.c[1].wait()

        def in_copy(i, slot):
            return _Pair(_in_half, i, slot)

        def out_copy(i, slot):
            return _Pair(_out_half, i, slot)

        for k in range(_NB - 1):
            in_copy(k, k).start()

        def step(i, _):
            slot = lax.rem(i, _NB)
            nxt = i + _NB - 1

            @pl.when(nxt < _NCH)
            def _():
                in_copy(nxt, lax.rem(nxt, _NB)).start()

            in_copy(i, slot).wait()

            @pl.when(i >= _NB)
            def _():
                out_copy(i - _NB, slot).wait()

            s = s_ref[pl.ds(pl.multiple_of(i * _CH, _CH), _CH)].reshape(_CH, 1)
            o = o_ref[pl.ds(pl.multiple_of(i * _CH, _CH), _CH)].reshape(_CH, 1)
            out_buf[slot] = jnp.clip(s * in_buf[slot] + o, 0.0, 1.0)
            out_copy(i, slot).start()
            return 0

        lax.fori_loop(0, _NCH, step, 0)
        for k in range(_NB):
            i = _NCH - _NB + k
            out_copy(i, i % _NB).wait()

    return pl.pallas_call(
        body,
        in_specs=[
            pl.BlockSpec(memory_space=pltpu.VMEM),
            pl.BlockSpec(memory_space=pltpu.VMEM),
            pl.BlockSpec(memory_space=pl.ANY),
        ],
        out_specs=pl.BlockSpec(memory_space=pl.ANY),
        out_shape=jax.ShapeDtypeStruct((_BATCH, _PIXELS), jnp.float32),
        scratch_shapes=[
            pltpu.VMEM((_NB, _CH, _PIXELS), jnp.float32),
            pltpu.VMEM((_NB, _CH, _PIXELS), jnp.float32),
            pltpu.SemaphoreType.DMA((_NB,)),
            pltpu.SemaphoreType.DMA((_NB,)),
        ],
        compiler_params=pltpu.CompilerParams(
            vmem_limit_bytes=62 * 1024 * 1024),
    )(scale, offset, image)


def kernel(idx, image, exposure_a, exposure_b):
    scale, offset = _sc_gather(idx, exposure_a.reshape(-1),
                               exposure_b.reshape(-1))
    return _tc_apply(scale, offset, image)
